# baseline (speedup 1.0000x reference)
import jax
import jax.numpy as jnp
from jax.experimental import pallas as pl
from jax.experimental.pallas import tpu as pltpu

_BM = 256
_NBUF = 4


def _gcn_body(x_ref, w_ref, adj_hbm, out_ref, s_vmem, bufs, sems):
    i = pl.program_id(0)
    n_steps = pl.num_programs(0)

    def _start(t):
        pltpu.make_async_copy(
            adj_hbm.at[pl.ds(t * _BM, _BM), :],
            bufs.at[t % _NBUF],
            sems.at[t % _NBUF],
        ).start()

    @pl.when(i == 0)
    def _():
        w = w_ref[...]
        for t in range(_NBUF):
            _start(t)
        s0 = jnp.dot(x_ref[0], w, preferred_element_type=jnp.float32)
        s1 = jnp.dot(x_ref[1], w, preferred_element_type=jnp.float32)
        s_vmem[...] = jnp.concatenate([s0, s1], axis=1)

    pltpu.make_async_copy(
        adj_hbm.at[pl.ds(i * _BM, _BM), :],
        bufs.at[i % _NBUF],
        sems.at[i % _NBUF],
    ).wait()

    acc = jnp.dot(bufs[i % _NBUF], s_vmem[...], preferred_element_type=jnp.float32)
    out_ref[...] = jnp.maximum(acc, 0.0)

    @pl.when(i + _NBUF < n_steps)
    def _():
        _start(i + _NBUF)


def kernel(input, adj, W):
    bs, n_agents, in_f = input.shape
    out_f = W.shape[1]

    grid = (n_agents // _BM,)
    out = pl.pallas_call(
        _gcn_body,
        grid=grid,
        in_specs=[
            pl.BlockSpec((bs, n_agents, in_f), lambda i: (0, 0, 0)),
            pl.BlockSpec((in_f, out_f), lambda i: (0, 0)),
            pl.BlockSpec(memory_space=pl.ANY),
        ],
        out_specs=pl.BlockSpec((_BM, bs * out_f), lambda i: (i, 0)),
        out_shape=jax.ShapeDtypeStruct((n_agents, bs * out_f), jnp.float32),
        scratch_shapes=[
            pltpu.VMEM((n_agents, bs * out_f), jnp.float32),
            pltpu.VMEM((_NBUF, _BM, n_agents), jnp.float32),
            pltpu.SemaphoreType.DMA((_NBUF,)),
        ],
        compiler_params=pltpu.CompilerParams(
            dimension_semantics=("arbitrary",),
            vmem_limit_bytes=120 * 1024 * 1024,
        ),
    )(input, W, adj)

    out = out.reshape(n_agents, bs, out_f).transpose(1, 0, 2)
    return out.reshape(bs * n_agents, out_f)
